# Initial kernel scaffold; baseline (speedup 1.0000x reference)
import jax
import jax.numpy as jnp
from jax.experimental import pallas as pl


def kernel(xs, embed, Wa_w, Wa_b, ua, path_map):
    raise NotImplementedError("write your pallas kernel here")


# speedup vs baseline: 6.4120x; 6.4120x over previous
import functools

import jax
import jax.numpy as jnp
from jax import lax
from jax.experimental import pallas as pl
from jax.experimental.pallas import tpu as pltpu
from jax.experimental.pallas import tpu_sc as plsc

_V = 100
_E = 128
_P = 6
_R = 64
_VP = 128
_NW = 32
_BT = 8


def _fused(xs, embed, Wa_w, ua, path_map):
    B, L = xs.shape
    rows_pw = B // _NW
    G = 4
    ngr = rows_pw // G
    tok_pg = G * L
    nblk = -(-tok_pg // 16)
    mesh = plsc.VectorSubcoreMesh(core_axis_name="c", subcore_axis_name="s")

    @functools.partial(
        pl.kernel, mesh=mesh,
        out_type=jax.ShapeDtypeStruct((B, L, _E), jnp.float32),
        compiler_params=pltpu.CompilerParams(needs_layout_passes=False),
        scratch_types=[
            pltpu.VMEM((rows_pw, L), jnp.int32),
            pltpu.VMEM((_VP, _E), jnp.float32),
            pltpu.VMEM((_V, _E), jnp.float32),
            pltpu.VMEM((_R, 2 * _E), jnp.float32),
            pltpu.VMEM((1, 1, 1, _R), jnp.float32),
            pltpu.VMEM((2 * _E,), jnp.float32),
            pltpu.VMEM((16, _E), jnp.float32),
            pltpu.VMEM((_V, _P), jnp.int32),
            pltpu.VMEM((G * L, _E), jnp.float32),
            pltpu.VMEM((G * L, _E), jnp.float32),
            pltpu.VMEM_SHARED((_BT, 16, _E), jnp.float32),
            pltpu.SemaphoreType.DMA,
            pltpu.SemaphoreType.DMA,
        ],
    )
    def k(emb_hbm, waw_hbm, ua_hbm, pm_hbm, idx_hbm, out_hbm,
          idx_v, tbl_v, emb_v, waw_v, ua_v, w12_v, gpart_v, pm_v, buf0, buf1,
          shared, wsem0, wsem1):
        cid = lax.axis_index("c")
        sid = lax.axis_index("s")
        wid = sid * 2 + cid
        base = wid * rows_pw
        lanes = lax.broadcasted_iota(jnp.int32, (16,), 0)
        zeros16 = jnp.zeros((16,), jnp.float32)

        pltpu.sync_copy(idx_hbm.at[pl.ds(base, rows_pw)], idx_v)
        pltpu.sync_copy(emb_hbm, emb_v)
        pltpu.sync_copy(pm_hbm, pm_v)
        pltpu.sync_copy(ua_hbm, ua_v)
        pltpu.sync_copy(waw_hbm, waw_v)

        def full16(x):
            return jnp.full((16,), x, jnp.int32)

        z16 = full16(0)

        @pl.when(sid < _BT)
        def _build():
            for cb in range(2 * _E // 16):
                def wbody(r, acc, cb=cb):
                    uar = plsc.load_gather(ua_v, [z16, z16, z16, full16(0) + r])
                    wrow = plsc.load_gather(waw_v, [full16(0) + r,
                                                    cb * 16 + lanes])
                    return acc + uar * wrow
                w12_v[pl.ds(cb * 16, 16)] = lax.fori_loop(0, _R, wbody, zeros16)

            vv = sid * 16 + lanes
            vvc = jnp.minimum(vv, _V - 1)
            pj = [plsc.load_gather(pm_v, [vvc, full16(p)]) for p in range(_P)]

            def sbody(c, carry):
                cc = full16(0) + c
                w1c = plsc.load_gather(w12_v, [cc])
                w2c = plsc.load_gather(w12_v, [cc + _E])
                ei = plsc.load_gather(emb_v, [vvc, cc])
                out = []
                for p in range(_P):
                    ej = plsc.load_gather(emb_v, [pj[p], cc])
                    s = carry[2 * p] + jnp.where(ej != 0.0, ei, 0.0) * w1c \
                        + ej * w2c
                    ss = carry[2 * p + 1] + ej
                    out += [s, ss]
                return tuple(out)

            carry = lax.fori_loop(0, _E, sbody, (zeros16,) * (2 * _P))
            neg = jnp.full((16,), -1e10, jnp.float32)
            scores = [jnp.where(carry[2 * p + 1] == 0.0, neg, carry[2 * p])
                      for p in range(_P)]
            m = scores[0]
            for s in scores[1:]:
                m = jnp.maximum(m, s)
            es = [jnp.exp(s - m) for s in scores]
            z = es[0]
            for e in es[1:]:
                z = z + e
            alpha = [e / z for e in es]

            def gbody(c, _):
                cc = full16(0) + c
                g = zeros16
                for p in range(_P):
                    g = g + alpha[p] * plsc.load_gather(emb_v, [pj[p], cc])
                plsc.store_scatter(gpart_v, [lanes, cc], g)
                return 0

            lax.fori_loop(0, _E, gbody, 0)
            pltpu.sync_copy(gpart_v, shared.at[sid])

        plsc.subcore_barrier()
        for t in range(_BT):
            pltpu.sync_copy(shared.at[t], tbl_v.at[pl.ds(t * 16, 16)])

        bufs = (buf0, buf1)
        wsems = (wsem0, wsem1)

        def write_start(g, b):
            for q in range(G):
                pltpu.async_copy(bufs[b].at[pl.ds(q * L, L)],
                                 out_hbm.at[base + g * G + q], wsems[b])

        def write_wait(g, b):
            for q in range(G):
                pltpu.make_async_copy(bufs[b].at[pl.ds(q * L, L)],
                                      out_hbm.at[base + g * G + q],
                                      wsems[b]).wait()

        for g in range(ngr):
            b = g % 2
            if g >= 2:
                write_wait(g - 2, b)

            def body(i, _, g=g, b=b):
                tok = i * 16 + lanes
                msk = tok < tok_pg
                tokc = jnp.minimum(tok, tok_pg - 1)
                gtok = g * tok_pg + tokc
                d0 = gtok // L
                d1 = gtok - d0 * L
                rowv = plsc.load_gather(idx_v, [d0, d1])
                for c in range(_E):
                    cc = full16(c)
                    val = plsc.load_gather(tbl_v, [rowv, cc])
                    plsc.store_scatter(bufs[b], [tokc, cc], val, mask=msk)
                return 0

            lax.fori_loop(0, nblk, body, 0)
            write_start(g, b)
        write_wait(ngr - 2, ngr % 2)
        write_wait(ngr - 1, (ngr - 1) % 2)

    return k(embed, Wa_w, ua, path_map, xs)


def kernel(xs, embed, Wa_w, Wa_b, ua, path_map):
    del Wa_b
    return _fused(xs, embed, Wa_w, ua, path_map)
